# baseline (speedup 1.0000x reference)
import jax
import jax.numpy as jnp
import numpy as np
from jax import lax
from jax.experimental import pallas as pl
from jax.experimental.pallas import tpu as pltpu

_S = 2048
_N = 4096
_D = 256
_C = 65
_R = 2048
_BIG = np.float32(1e30)


def _ce(a, b):
    return jnp.minimum(a, b), jnp.maximum(a, b)


def _merge22(a, b):
    l0, h0 = _ce(a[0], b[0])
    l1, h1 = _ce(a[1], b[1])
    m0, m1 = _ce(h0, l1)
    return [l0, m0, m1, h1]


def _merge44_bot5(a, b):
    e = _merge22([a[0], a[2]], [b[0], b[2]])
    o0, oh = _ce(a[1], b[1])
    o1 = jnp.minimum(oh, jnp.minimum(a[3], b[3]))
    r1, r2 = _ce(e[1], o0)
    r3, r4 = _ce(e[2], o1)
    return [e[0], r1, r2, r3, r4]


def _merge55_bot5(a, b):
    m0 = jnp.minimum(a[0], b[0])
    m1 = jnp.minimum(jnp.minimum(a[1], b[1]), jnp.maximum(a[0], b[0]))
    m2 = jnp.minimum(jnp.minimum(a[2], b[2]),
                     jnp.minimum(jnp.maximum(a[0], b[1]),
                                 jnp.maximum(a[1], b[0])))
    m3 = jnp.minimum(jnp.minimum(a[3], b[3]),
                     jnp.minimum(jnp.maximum(a[0], b[2]),
                                 jnp.minimum(jnp.maximum(a[1], b[1]),
                                             jnp.maximum(a[2], b[0]))))
    m4 = jnp.minimum(jnp.minimum(a[4], b[4]),
                     jnp.minimum(jnp.minimum(jnp.maximum(a[0], b[3]),
                                             jnp.maximum(a[1], b[2])),
                                 jnp.minimum(jnp.maximum(a[2], b[1]),
                                             jnp.maximum(a[3], b[0]))))
    return [m0, m1, m2, m3, m4]


def _row_top5(x):
    nchunks = x.shape[1] // 128
    c = [x[:, j * 128:(j + 1) * 128] for j in range(nchunks)]
    pairs = [_ce(c[2 * j], c[2 * j + 1]) for j in range(8)]
    quads = [_merge22(pairs[2 * j], pairs[2 * j + 1]) for j in range(4)]
    t0 = _merge44_bot5(quads[0], quads[1])
    t1 = _merge44_bot5(quads[2], quads[3])
    l0, l1, l2, l3, l4 = _merge55_bot5(t0, t1)

    lane = lax.broadcasted_iota(jnp.int32, l0.shape, 1).astype(jnp.float32)
    d1 = None
    m = None
    for t in range(5):
        m = jnp.min(l0, axis=1)
        if t == 0:
            d1 = m
        if t < 4:
            candlane = jnp.where(l0 == m[:, None], lane, _BIG)
            j = jnp.min(candlane, axis=1)
            hit = lane == j[:, None]
            l0 = jnp.where(hit, l1, l0)
            l1 = jnp.where(hit, l2, l1)
            l2 = jnp.where(hit, l3, l2)
            l3 = jnp.where(hit, l4, l3)
    return d1, m


def _body(ffull_ref, sq_ref, logit_ref, y_ref,
          topk_ref, cl_ref, ce_ref, acc_ref):
    i = pl.program_id(0)
    nsteps = pl.num_programs(0)

    frow = ffull_ref[pl.ds(i * _R, _R), :]
    sq_i = sq_ref[0, pl.ds(i * _R, _R)]
    row = i * _R + lax.broadcasted_iota(jnp.int32, (_R, _S), 0)
    col = lax.broadcasted_iota(jnp.int32, (_R, _S), 1)

    def half_top5(h):
        fcol = ffull_ref[pl.ds(h * _S, _S), :]
        scores = lax.dot_general(frow, fcol, (((1,), (1,)), ((), ())),
                                 preferred_element_type=jnp.float32)
        sq_j = sq_ref[0, pl.ds(h * _S, _S)]
        d2 = (sq_i[:, None] + sq_j[None, :]) - 2.0 * scores
        d2 = jnp.where(col + h * _S == row, _BIG, d2)
        return _row_top5(d2)

    d1s, d5s = half_top5(0)
    d1t, d5t = half_top5(1)
    topk_ref[...] = jnp.stack([d1s, d5s, d1t, d5t], axis=0)

    lg = logit_ref[...]
    mx = jnp.max(lg, axis=1)
    e = jnp.exp(lg - mx[:, None])
    z = jnp.sum(e, axis=1)

    @pl.when(i == 0)
    def _():
        acc_ref[0] = 0.0
        acc_ref[1] = 0.0

    @pl.when(i < _S // _R)
    def _():
        y = y_ref[0, 0, :]
        ci = lax.broadcasted_iota(jnp.int32, (_R, _C), 1)
        picked = jnp.sum(jnp.where(ci == y[:, None], lg, 0.0), axis=1)
        lse = mx + jnp.log(z)
        acc_ref[0] += jnp.sum(lse - picked)

    @pl.when(i >= _S // _R)
    def _():
        p = e / z[:, None]
        h = jnp.sum(-p * jnp.log2(jnp.maximum(p, 1e-8)), axis=1)
        acc_ref[1] += jnp.sum(h)

    @pl.when(i == nsteps - 1)
    def _():
        cl_ref[...] = (acc_ref[0] / jnp.float32(_S)).reshape(1, 1)
        ce_ref[...] = (acc_ref[1] / jnp.float32(_N - _S)).reshape(1, 1)


def _entropy_like_ref(p, axis=1):
    return jnp.mean(jnp.sum(-p * jnp.log2(jnp.clip(p, 1e-08)), axis=axis))


def kernel(Feature_all, logit_all, y_source, device, k):
    f = Feature_all.astype(jnp.float32)
    lg = logit_all.astype(jnp.float32)
    y3 = y_source.astype(jnp.int32).reshape(_S // _R, 1, _R)
    sq = jnp.sum(f * f, axis=1).reshape(1, _N)

    topk, cl, ce = pl.pallas_call(
        _body,
        grid=(_N // _R,),
        in_specs=[
            pl.BlockSpec((_N, _D), lambda i: (0, 0)),
            pl.BlockSpec((1, _N), lambda i: (0, 0)),
            pl.BlockSpec((_R, _C), lambda i: (i, 0)),
            pl.BlockSpec((1, 1, _R), lambda i: (jnp.minimum(i, _S // _R - 1), 0, 0)),
        ],
        out_specs=[
            pl.BlockSpec((4, _R), lambda i: (0, i)),
            pl.BlockSpec((1, 1), lambda i: (0, 0)),
            pl.BlockSpec((1, 1), lambda i: (0, 0)),
        ],
        out_shape=[
            jax.ShapeDtypeStruct((4, _N), jnp.float32),
            jax.ShapeDtypeStruct((1, 1), jnp.float32),
            jax.ShapeDtypeStruct((1, 1), jnp.float32),
        ],
        scratch_shapes=[
            pltpu.SMEM((2,), jnp.float32),
        ],
    )(f, sq, lg, y3)

    dist1s = jnp.sqrt(jnp.clip(topk[0], 0.0))
    v_s = jnp.clip(jnp.sqrt(jnp.clip(topk[1], 0.0)), 1e-08)
    dist1t = jnp.sqrt(jnp.clip(topk[2], 0.0))
    v_t = jnp.clip(jnp.sqrt(jnp.clip(topk[3], 0.0)), 1e-08)
    sim_s = jnp.exp(-dist1s / (2.0 * v_s ** 2))
    sim_t = jnp.exp(-dist1t / (2.0 * v_t ** 2))

    si = sim_s[:_S]
    se = sim_t[:_S]
    ti = sim_t[_S:]
    te = sim_s[_S:]
    sp = si / (si + se)
    tp = ti / (ti + te)
    sprobs = jnp.stack([sp, 1.0 - sp], axis=1)
    tprobs = jnp.stack([tp, 1.0 - tp], axis=1)
    max_entropy = jnp.log2(jnp.asarray(2.0)) * 2.0
    loss_reg = max_entropy - (_entropy_like_ref(sprobs) + _entropy_like_ref(tprobs))
    return (cl[0, 0], ce[0, 0], loss_reg)


# speedup vs baseline: 1.0040x; 1.0040x over previous
import jax
import jax.numpy as jnp
import numpy as np
from jax import lax
from jax.experimental import pallas as pl
from jax.experimental.pallas import tpu as pltpu

_S = 2048
_N = 4096
_D = 256
_C = 65
_R = 2048
_BIG = np.float32(1e30)


def _ce(a, b):
    return jnp.minimum(a, b), jnp.maximum(a, b)


def _merge22(a, b):
    l0, h0 = _ce(a[0], b[0])
    l1, h1 = _ce(a[1], b[1])
    m0, m1 = _ce(h0, l1)
    return [l0, m0, m1, h1]


def _merge44_bot5(a, b):
    e = _merge22([a[0], a[2]], [b[0], b[2]])
    o0, oh = _ce(a[1], b[1])
    o1 = jnp.minimum(oh, jnp.minimum(a[3], b[3]))
    r1, r2 = _ce(e[1], o0)
    r3, r4 = _ce(e[2], o1)
    return [e[0], r1, r2, r3, r4]


def _merge55_bot5(a, b):
    m0 = jnp.minimum(a[0], b[0])
    m1 = jnp.minimum(jnp.minimum(a[1], b[1]), jnp.maximum(a[0], b[0]))
    m2 = jnp.minimum(jnp.minimum(a[2], b[2]),
                     jnp.minimum(jnp.maximum(a[0], b[1]),
                                 jnp.maximum(a[1], b[0])))
    m3 = jnp.minimum(jnp.minimum(a[3], b[3]),
                     jnp.minimum(jnp.maximum(a[0], b[2]),
                                 jnp.minimum(jnp.maximum(a[1], b[1]),
                                             jnp.maximum(a[2], b[0]))))
    m4 = jnp.minimum(jnp.minimum(a[4], b[4]),
                     jnp.minimum(jnp.minimum(jnp.maximum(a[0], b[3]),
                                             jnp.maximum(a[1], b[2])),
                                 jnp.minimum(jnp.maximum(a[2], b[1]),
                                             jnp.maximum(a[3], b[0]))))
    return [m0, m1, m2, m3, m4]


def _row_top5(x):
    nchunks = x.shape[1] // 128
    c = [x[:, j * 128:(j + 1) * 128] for j in range(nchunks)]
    pairs = [_ce(c[2 * j], c[2 * j + 1]) for j in range(8)]
    quads = [_merge22(pairs[2 * j], pairs[2 * j + 1]) for j in range(4)]
    t0 = _merge44_bot5(quads[0], quads[1])
    t1 = _merge44_bot5(quads[2], quads[3])
    l0, l1, l2, l3, l4 = _merge55_bot5(t0, t1)

    lane = lax.broadcasted_iota(jnp.int32, l0.shape, 1).astype(jnp.float32)
    d1 = None
    m = None
    for t in range(5):
        m = jnp.min(l0, axis=1)
        if t == 0:
            d1 = m
        if t < 4:
            candlane = jnp.where(l0 == m[:, None], lane, _BIG)
            j = jnp.min(candlane, axis=1)
            hit = lane == j[:, None]
            l0 = jnp.where(hit, l1, l0)
            if t < 3:
                l1 = jnp.where(hit, l2, l1)
            if t < 2:
                l2 = jnp.where(hit, l3, l2)
            if t < 1:
                l3 = jnp.where(hit, l4, l3)
    return d1, m


def _body(ffull_ref, sq_ref, logit_ref, y_ref,
          topk_ref, cl_ref, ce_ref, acc_ref):
    i = pl.program_id(0)
    nsteps = pl.num_programs(0)

    frow = ffull_ref[pl.ds(i * _R, _R), :]
    sq_i = sq_ref[0, pl.ds(i * _R, _R)]
    row = i * _R + lax.broadcasted_iota(jnp.int32, (_R, _S), 0)
    col = lax.broadcasted_iota(jnp.int32, (_R, _S), 1)

    def half_top5(h):
        fcol = ffull_ref[pl.ds(h * _S, _S), :]
        scores = lax.dot_general(frow, fcol, (((1,), (1,)), ((), ())),
                                 preferred_element_type=jnp.float32)
        sq_j = sq_ref[0, pl.ds(h * _S, _S)]
        d2 = (sq_i[:, None] + sq_j[None, :]) - 2.0 * scores
        d2 = jnp.where(col + h * _S == row, _BIG, d2)
        return _row_top5(d2)

    d1s, d5s = half_top5(0)
    d1t, d5t = half_top5(1)
    topk_ref[...] = jnp.stack([d1s, d5s, d1t, d5t], axis=0)

    lg = logit_ref[...]
    mx = jnp.max(lg, axis=1)
    e = jnp.exp(lg - mx[:, None])
    z = jnp.sum(e, axis=1)

    @pl.when(i == 0)
    def _():
        acc_ref[0] = 0.0
        acc_ref[1] = 0.0

    @pl.when(i < _S // _R)
    def _():
        y = y_ref[0, 0, :]
        ci = lax.broadcasted_iota(jnp.int32, (_R, _C), 1)
        picked = jnp.sum(jnp.where(ci == y[:, None], lg, 0.0), axis=1)
        lse = mx + jnp.log(z)
        acc_ref[0] += jnp.sum(lse - picked)

    @pl.when(i >= _S // _R)
    def _():
        p = e / z[:, None]
        h = jnp.sum(-p * jnp.log2(jnp.maximum(p, 1e-8)), axis=1)
        acc_ref[1] += jnp.sum(h)

    @pl.when(i == nsteps - 1)
    def _():
        cl_ref[...] = (acc_ref[0] / jnp.float32(_S)).reshape(1, 1)
        ce_ref[...] = (acc_ref[1] / jnp.float32(_N - _S)).reshape(1, 1)


def _entropy_like_ref(p, axis=1):
    return jnp.mean(jnp.sum(-p * jnp.log2(jnp.clip(p, 1e-08)), axis=axis))


def kernel(Feature_all, logit_all, y_source, device, k):
    f = Feature_all.astype(jnp.float32)
    lg = logit_all.astype(jnp.float32)
    y3 = y_source.astype(jnp.int32).reshape(_S // _R, 1, _R)
    sq = jnp.sum(f * f, axis=1).reshape(1, _N)

    topk, cl, ce = pl.pallas_call(
        _body,
        grid=(_N // _R,),
        in_specs=[
            pl.BlockSpec((_N, _D), lambda i: (0, 0)),
            pl.BlockSpec((1, _N), lambda i: (0, 0)),
            pl.BlockSpec((_R, _C), lambda i: (i, 0)),
            pl.BlockSpec((1, 1, _R), lambda i: (jnp.minimum(i, _S // _R - 1), 0, 0)),
        ],
        out_specs=[
            pl.BlockSpec((4, _R), lambda i: (0, i)),
            pl.BlockSpec((1, 1), lambda i: (0, 0)),
            pl.BlockSpec((1, 1), lambda i: (0, 0)),
        ],
        out_shape=[
            jax.ShapeDtypeStruct((4, _N), jnp.float32),
            jax.ShapeDtypeStruct((1, 1), jnp.float32),
            jax.ShapeDtypeStruct((1, 1), jnp.float32),
        ],
        scratch_shapes=[
            pltpu.SMEM((2,), jnp.float32),
        ],
    )(f, sq, lg, y3)

    dist1s = jnp.sqrt(jnp.clip(topk[0], 0.0))
    v_s = jnp.clip(jnp.sqrt(jnp.clip(topk[1], 0.0)), 1e-08)
    dist1t = jnp.sqrt(jnp.clip(topk[2], 0.0))
    v_t = jnp.clip(jnp.sqrt(jnp.clip(topk[3], 0.0)), 1e-08)
    sim_s = jnp.exp(-dist1s / (2.0 * v_s ** 2))
    sim_t = jnp.exp(-dist1t / (2.0 * v_t ** 2))

    si = sim_s[:_S]
    se = sim_t[:_S]
    ti = sim_t[_S:]
    te = sim_s[_S:]
    sp = si / (si + se)
    tp = ti / (ti + te)
    sprobs = jnp.stack([sp, 1.0 - sp], axis=1)
    tprobs = jnp.stack([tp, 1.0 - tp], axis=1)
    max_entropy = jnp.log2(jnp.asarray(2.0)) * 2.0
    loss_reg = max_entropy - (_entropy_like_ref(sprobs) + _entropy_like_ref(tprobs))
    return (cl[0, 0], ce[0, 0], loss_reg)


# speedup vs baseline: 1.0815x; 1.0772x over previous
import jax
import jax.numpy as jnp
import numpy as np
from jax import lax
from jax.experimental import pallas as pl
from jax.experimental.pallas import tpu as pltpu

_S = 2048
_N = 4096
_D = 256
_C = 65
_R = 2048
_BIG = np.float32(1e30)


def _ce(a, b):
    return jnp.minimum(a, b), jnp.maximum(a, b)


def _merge22(a, b):
    l0, h0 = _ce(a[0], b[0])
    l1, h1 = _ce(a[1], b[1])
    m0, m1 = _ce(h0, l1)
    return [l0, m0, m1, h1]


def _merge44_bot5(a, b):
    e = _merge22([a[0], a[2]], [b[0], b[2]])
    o0, oh = _ce(a[1], b[1])
    o1 = jnp.minimum(oh, jnp.minimum(a[3], b[3]))
    r1, r2 = _ce(e[1], o0)
    r3, r4 = _ce(e[2], o1)
    return [e[0], r1, r2, r3, r4]


def _merge55_bot5(a, b):
    m0 = jnp.minimum(a[0], b[0])
    m1 = jnp.minimum(jnp.minimum(a[1], b[1]), jnp.maximum(a[0], b[0]))
    m2 = jnp.minimum(jnp.minimum(a[2], b[2]),
                     jnp.minimum(jnp.maximum(a[0], b[1]),
                                 jnp.maximum(a[1], b[0])))
    m3 = jnp.minimum(jnp.minimum(a[3], b[3]),
                     jnp.minimum(jnp.maximum(a[0], b[2]),
                                 jnp.minimum(jnp.maximum(a[1], b[1]),
                                             jnp.maximum(a[2], b[0]))))
    m4 = jnp.minimum(jnp.minimum(a[4], b[4]),
                     jnp.minimum(jnp.minimum(jnp.maximum(a[0], b[3]),
                                             jnp.maximum(a[1], b[2])),
                                 jnp.minimum(jnp.maximum(a[2], b[1]),
                                             jnp.maximum(a[3], b[0]))))
    return [m0, m1, m2, m3, m4]


def _row_top5(x):
    nchunks = x.shape[1] // 128
    c = [x[:, j * 128:(j + 1) * 128] for j in range(nchunks)]
    pairs = [_ce(c[2 * j], c[2 * j + 1]) for j in range(8)]
    quads = [_merge22(pairs[2 * j], pairs[2 * j + 1]) for j in range(4)]
    t0 = _merge44_bot5(quads[0], quads[1])
    t1 = _merge44_bot5(quads[2], quads[3])
    l0, l1, l2, l3, l4 = _merge55_bot5(t0, t1)

    lane = lax.broadcasted_iota(jnp.int32, l0.shape, 1).astype(jnp.float32)
    d1 = None
    m = None
    for t in range(5):
        m = jnp.min(l0, axis=1)
        if t == 0:
            d1 = m
        if t < 4:
            candlane = jnp.where(l0 == m[:, None], lane, _BIG)
            j = jnp.min(candlane, axis=1)
            hit = lane == j[:, None]
            l0 = jnp.where(hit, l1, l0)
            if t < 3:
                l1 = jnp.where(hit, l2, l1)
            if t < 2:
                l2 = jnp.where(hit, l3, l2)
            if t < 1:
                l3 = jnp.where(hit, l4, l3)
    return d1, m


def _body(ffull_ref, sq_ref, logit_ref, y_ref,
          topk_ref, cl_ref, ce_ref, acc_ref):
    i = pl.program_id(0)
    nsteps = pl.num_programs(0)

    frow = ffull_ref[pl.ds(i * _R, _R), :]
    sq_i = sq_ref[0, pl.ds(i * _R, _R)]
    assert _R == _S
    diag = (lax.broadcasted_iota(jnp.int32, (_R, _S), 0)
            == lax.broadcasted_iota(jnp.int32, (_R, _S), 1))

    def half_top5(h):
        fcol = ffull_ref[pl.ds(h * _S, _S), :]
        scores = lax.dot_general(frow, fcol, (((1,), (1,)), ((), ())),
                                 preferred_element_type=jnp.float32)
        sq_j = sq_ref[0, pl.ds(h * _S, _S)]
        d2 = (sq_i[:, None] + sq_j[None, :]) - 2.0 * scores
        d2 = jnp.where(jnp.logical_and(i == h, diag), _BIG, d2)
        return _row_top5(d2)

    d1s, d5s = half_top5(0)
    d1t, d5t = half_top5(1)
    topk_ref[...] = jnp.stack([d1s, d5s, d1t, d5t], axis=0)

    lg = logit_ref[...]
    mx = jnp.max(lg, axis=1)
    e = jnp.exp(lg - mx[:, None])
    z = jnp.sum(e, axis=1)

    @pl.when(i == 0)
    def _():
        acc_ref[0] = 0.0
        acc_ref[1] = 0.0

    @pl.when(i < _S // _R)
    def _():
        y = y_ref[0, 0, :]
        ci = lax.broadcasted_iota(jnp.int32, (_R, _C), 1)
        picked = jnp.sum(jnp.where(ci == y[:, None], lg, 0.0), axis=1)
        lse = mx + jnp.log(z)
        acc_ref[0] += jnp.sum(lse - picked)

    @pl.when(i >= _S // _R)
    def _():
        p = e / z[:, None]
        h = jnp.sum(-p * jnp.log2(jnp.maximum(p, 1e-8)), axis=1)
        acc_ref[1] += jnp.sum(h)

    @pl.when(i == nsteps - 1)
    def _():
        cl_ref[...] = (acc_ref[0] / jnp.float32(_S)).reshape(1, 1)
        ce_ref[...] = (acc_ref[1] / jnp.float32(_N - _S)).reshape(1, 1)


def _entropy_like_ref(p, axis=1):
    return jnp.mean(jnp.sum(-p * jnp.log2(jnp.clip(p, 1e-08)), axis=axis))


def kernel(Feature_all, logit_all, y_source, device, k):
    f = Feature_all.astype(jnp.float32)
    lg = logit_all.astype(jnp.float32)
    y3 = y_source.astype(jnp.int32).reshape(_S // _R, 1, _R)
    sq = jnp.sum(f * f, axis=1).reshape(1, _N)

    topk, cl, ce = pl.pallas_call(
        _body,
        grid=(_N // _R,),
        in_specs=[
            pl.BlockSpec((_N, _D), lambda i: (0, 0)),
            pl.BlockSpec((1, _N), lambda i: (0, 0)),
            pl.BlockSpec((_R, _C), lambda i: (i, 0)),
            pl.BlockSpec((1, 1, _R), lambda i: (jnp.minimum(i, _S // _R - 1), 0, 0)),
        ],
        out_specs=[
            pl.BlockSpec((4, _R), lambda i: (0, i)),
            pl.BlockSpec((1, 1), lambda i: (0, 0)),
            pl.BlockSpec((1, 1), lambda i: (0, 0)),
        ],
        out_shape=[
            jax.ShapeDtypeStruct((4, _N), jnp.float32),
            jax.ShapeDtypeStruct((1, 1), jnp.float32),
            jax.ShapeDtypeStruct((1, 1), jnp.float32),
        ],
        scratch_shapes=[
            pltpu.SMEM((2,), jnp.float32),
        ],
    )(f, sq, lg, y3)

    dist1s = jnp.sqrt(jnp.clip(topk[0], 0.0))
    v_s = jnp.clip(jnp.sqrt(jnp.clip(topk[1], 0.0)), 1e-08)
    dist1t = jnp.sqrt(jnp.clip(topk[2], 0.0))
    v_t = jnp.clip(jnp.sqrt(jnp.clip(topk[3], 0.0)), 1e-08)
    sim_s = jnp.exp(-dist1s / (2.0 * v_s ** 2))
    sim_t = jnp.exp(-dist1t / (2.0 * v_t ** 2))

    si = sim_s[:_S]
    se = sim_t[:_S]
    ti = sim_t[_S:]
    te = sim_s[_S:]
    sp = si / (si + se)
    tp = ti / (ti + te)
    sprobs = jnp.stack([sp, 1.0 - sp], axis=1)
    tprobs = jnp.stack([tp, 1.0 - tp], axis=1)
    max_entropy = jnp.log2(jnp.asarray(2.0)) * 2.0
    loss_reg = max_entropy - (_entropy_like_ref(sprobs) + _entropy_like_ref(tprobs))
    return (cl[0, 0], ce[0, 0], loss_reg)
